# Initial kernel scaffold; baseline (speedup 1.0000x reference)
import functools

import jax
import jax.numpy as jnp
from jax import lax
from jax.experimental import pallas as pl
from jax.experimental.pallas import tpu as pltpu
from jax.experimental.pallas import tpu_sc as plsc

D, H, W = 160, 192, 224
N = D * H * W
HW = H * W
NC, NS = 2, 16
NW = NC * NS
ROWS_PER_TILE = (N // W) // NW
CHUNK_ROWS = 10
CHUNK = CHUNK_ROWS * W
NCHUNK = ROWS_PER_TILE // CHUNK_ROWS
PER_TILE = ROWS_PER_TILE * W
VPR = W // 16


def _floor(c):
    t = c.astype(jnp.int32)
    tf = t.astype(jnp.float32)
    adj = tf > c
    fi = t - jnp.where(adj, 1, 0)
    ff = tf - jnp.where(adj, 1.0, 0.0)
    return fi, ff


def _corner_indices(cz, cy, cx):
    fz, _ = _floor(cz)
    fy, _ = _floor(cy)
    fx, _ = _floor(cx)
    z0 = jnp.clip(fz, 0, D - 1)
    z1 = jnp.clip(fz + 1, 0, D - 1)
    y0 = jnp.clip(fy, 0, H - 1)
    y1 = jnp.clip(fy + 1, 0, H - 1)
    x0 = jnp.clip(fx, 0, W - 1)
    x1 = jnp.clip(fx + 1, 0, W - 1)
    r00 = z0 * HW + y0 * W
    r01 = z0 * HW + y1 * W
    r10 = z1 * HW + y0 * W
    r11 = z1 * HW + y1 * W
    return (r00 + x0, r00 + x1, r01 + x0, r01 + x1,
            r10 + x0, r10 + x1, r11 + x0, r11 + x1)


def _dim_weights(c, size):
    fi, ff = _floor(c)
    fr = c - ff
    m0 = (fi >= 0) & (fi < size)
    m1 = (fi >= -1) & (fi < size - 1)
    w0 = jnp.where(m0, 1.0 - fr, 0.0)
    w1 = jnp.where(m1, fr, 0.0)
    return w0, w1


def _combine(cz, cy, cx, g):
    wz0, wz1 = _dim_weights(cz, D)
    wy0, wy1 = _dim_weights(cy, H)
    wx0, wx1 = _dim_weights(cx, W)
    w00 = wz0 * wy0
    w01 = wz0 * wy1
    w10 = wz1 * wy0
    w11 = wz1 * wy1
    return ((w00 * wx0) * g[0] + (w00 * wx1) * g[1] +
            (w01 * wx0) * g[2] + (w01 * wx1) * g[3] +
            (w10 * wx0) * g[4] + (w10 * wx1) * g[5] +
            (w11 * wx0) * g[6] + (w11 * wx1) * g[7])


def _body(src_hbm, flow_hbm, out_hbm, flow_v, idx_v, g_v, out_v, sem):
    wid = lax.axis_index("s") * NC + lax.axis_index("c")
    iota_f = lax.iota(jnp.int32, 16).astype(jnp.float32)
    xvecs = [iota_f + jnp.float32(16 * v) for v in range(VPR)]

    def chunk_body(ci, carry):
        base = wid * PER_TILE + ci * CHUNK
        pltpu.sync_copy(flow_hbm.at[0, pl.ds(base, CHUNK)], flow_v.at[0])
        pltpu.sync_copy(flow_hbm.at[1, pl.ds(base, CHUNK)], flow_v.at[1])
        pltpu.sync_copy(flow_hbm.at[2, pl.ds(base, CHUNK)], flow_v.at[2])
        grow0 = wid * ROWS_PER_TILE + ci * CHUNK_ROWS

        def row_idx(r, c2):
            grow = grow0 + r
            zf = (grow // H).astype(jnp.float32)
            yf = (grow % H).astype(jnp.float32)
            off = r * W
            for v in range(VPR):
                o = off + v * 16
                cz = flow_v[0, pl.ds(o, 16)] + zf
                cy = flow_v[1, pl.ds(o, 16)] + yf
                cx = flow_v[2, pl.ds(o, 16)] + xvecs[v]
                idx = _corner_indices(cz, cy, cx)
                for c in range(8):
                    idx_v[c, pl.ds(o, 16)] = idx[c]
            return c2

        lax.fori_loop(0, CHUNK_ROWS, row_idx, 0)

        copies = [pltpu.async_copy(src_hbm.at[idx_v.at[c]], g_v.at[c], sem)
                  for c in range(8)]
        for cp in copies:
            cp.wait()

        def row_comb(r, c2):
            grow = grow0 + r
            zf = (grow // H).astype(jnp.float32)
            yf = (grow % H).astype(jnp.float32)
            off = r * W
            for v in range(VPR):
                o = off + v * 16
                cz = flow_v[0, pl.ds(o, 16)] + zf
                cy = flow_v[1, pl.ds(o, 16)] + yf
                cx = flow_v[2, pl.ds(o, 16)] + xvecs[v]
                g = [g_v[c, pl.ds(o, 16)] for c in range(8)]
                out_v[pl.ds(o, 16)] = _combine(cz, cy, cx, g)
            return c2

        lax.fori_loop(0, CHUNK_ROWS, row_comb, 0)
        pltpu.sync_copy(out_v, out_hbm.at[pl.ds(base, CHUNK)])
        return carry

    lax.fori_loop(0, NCHUNK, chunk_body, 0)


@jax.jit
def _run(src_flat, flow_flat):
    mesh = plsc.VectorSubcoreMesh(core_axis_name="c", subcore_axis_name="s")
    f = functools.partial(
        pl.kernel,
        out_type=jax.ShapeDtypeStruct((N,), jnp.float32),
        mesh=mesh,
        scratch_types=[
            pltpu.VMEM((3, CHUNK), jnp.float32),
            pltpu.VMEM((8, CHUNK), jnp.int32),
            pltpu.VMEM((8, CHUNK), jnp.float32),
            pltpu.VMEM((CHUNK,), jnp.float32),
            pltpu.SemaphoreType.DMA,
        ],
    )(_body)
    return f(src_flat, flow_flat)


def kernel(source, flow_field):
    src_flat = source.reshape(N)
    flow_flat = flow_field.reshape(3, N)
    out = _run(src_flat, flow_flat)
    return out.reshape(source.shape)


# speedup vs baseline: 1.6523x; 1.6523x over previous
import functools

import jax
import jax.numpy as jnp
from jax import lax
from jax.experimental import pallas as pl
from jax.experimental.pallas import tpu as pltpu
from jax.experimental.pallas import tpu_sc as plsc

D, H, W = 160, 192, 224
N = D * H * W
HW = H * W
NC, NS = 2, 16
NW = NC * NS
ROWS_PER_TILE = (N // W) // NW
CHUNK_ROWS = 10
CHUNK = CHUNK_ROWS * W
NCHUNK = ROWS_PER_TILE // CHUNK_ROWS
PER_TILE = ROWS_PER_TILE * W
VPR = W // 16


def _floor(c):
    t = c.astype(jnp.int32)
    tf = t.astype(jnp.float32)
    adj = tf > c
    fi = t - jnp.where(adj, 1, 0)
    ff = tf - jnp.where(adj, 1.0, 0.0)
    return fi, ff


def _corner_indices(cz, cy, cx):
    fz, _ = _floor(cz)
    fy, _ = _floor(cy)
    fx, _ = _floor(cx)
    z0 = jnp.clip(fz, 0, D - 1)
    z1 = jnp.clip(fz + 1, 0, D - 1)
    y0 = jnp.clip(fy, 0, H - 1)
    y1 = jnp.clip(fy + 1, 0, H - 1)
    x0 = jnp.clip(fx, 0, W - 1)
    x1 = jnp.clip(fx + 1, 0, W - 1)
    r00 = z0 * HW + y0 * W
    r01 = z0 * HW + y1 * W
    r10 = z1 * HW + y0 * W
    r11 = z1 * HW + y1 * W
    return (r00 + x0, r00 + x1, r01 + x0, r01 + x1,
            r10 + x0, r10 + x1, r11 + x0, r11 + x1)


def _dim_weights(c, size):
    fi, ff = _floor(c)
    fr = c - ff
    m0 = (fi >= 0) & (fi < size)
    m1 = (fi >= -1) & (fi < size - 1)
    w0 = jnp.where(m0, 1.0 - fr, 0.0)
    w1 = jnp.where(m1, fr, 0.0)
    return w0, w1


def _combine(cz, cy, cx, g):
    wz0, wz1 = _dim_weights(cz, D)
    wy0, wy1 = _dim_weights(cy, H)
    wx0, wx1 = _dim_weights(cx, W)
    w00 = wz0 * wy0
    w01 = wz0 * wy1
    w10 = wz1 * wy0
    w11 = wz1 * wy1
    return ((w00 * wx0) * g[0] + (w00 * wx1) * g[1] +
            (w01 * wx0) * g[2] + (w01 * wx1) * g[3] +
            (w10 * wx0) * g[4] + (w10 * wx1) * g[5] +
            (w11 * wx0) * g[6] + (w11 * wx1) * g[7])


def _body(src_hbm, flow_hbm, out_hbm, fz_v, fy_v, fx_v, idx_refs, g_refs,
          out_v, sem):
    wid = lax.axis_index("s") * NC + lax.axis_index("c")
    iota_f = lax.iota(jnp.int32, 16).astype(jnp.float32)
    xvecs = [iota_f + jnp.float32(16 * v) for v in range(VPR)]

    def chunk_body(ci, carry):
        base = wid * PER_TILE + ci * CHUNK
        pltpu.sync_copy(flow_hbm.at[pl.ds(base, CHUNK)], fz_v)
        pltpu.sync_copy(flow_hbm.at[pl.ds(N + base, CHUNK)], fy_v)
        pltpu.sync_copy(flow_hbm.at[pl.ds(2 * N + base, CHUNK)], fx_v)
        grow0 = wid * ROWS_PER_TILE + ci * CHUNK_ROWS

        def row_idx(r, c2):
            grow = grow0 + r
            zf = (grow // H).astype(jnp.float32)
            yf = (grow % H).astype(jnp.float32)
            off = r * W
            for v in range(VPR):
                o = off + v * 16
                cz = fz_v[pl.ds(o, 16)] + zf
                cy = fy_v[pl.ds(o, 16)] + yf
                cx = fx_v[pl.ds(o, 16)] + xvecs[v]
                idx = _corner_indices(cz, cy, cx)
                for c in range(8):
                    idx_refs[c][pl.ds(o, 16)] = idx[c]
            return c2

        lax.fori_loop(0, CHUNK_ROWS, row_idx, 0)

        copies = [pltpu.async_copy(src_hbm.at[idx_refs[c]], g_refs[c], sem)
                  for c in range(8)]
        for cp in copies:
            cp.wait()

        def row_comb(r, c2):
            grow = grow0 + r
            zf = (grow // H).astype(jnp.float32)
            yf = (grow % H).astype(jnp.float32)
            off = r * W
            for v in range(VPR):
                o = off + v * 16
                cz = fz_v[pl.ds(o, 16)] + zf
                cy = fy_v[pl.ds(o, 16)] + yf
                cx = fx_v[pl.ds(o, 16)] + xvecs[v]
                g = [g_refs[c][pl.ds(o, 16)] for c in range(8)]
                out_v[pl.ds(o, 16)] = _combine(cz, cy, cx, g)
            return c2

        lax.fori_loop(0, CHUNK_ROWS, row_comb, 0)
        pltpu.sync_copy(out_v, out_hbm.at[pl.ds(base, CHUNK)])
        return carry

    lax.fori_loop(0, NCHUNK, chunk_body, 0)


@jax.jit
def _run(src_flat, flow_flat):
    mesh = plsc.VectorSubcoreMesh(core_axis_name="c", subcore_axis_name="s")
    f = functools.partial(
        pl.kernel,
        out_type=jax.ShapeDtypeStruct((N,), jnp.float32),
        mesh=mesh,
        scratch_types=[
            pltpu.VMEM((CHUNK,), jnp.float32),
            pltpu.VMEM((CHUNK,), jnp.float32),
            pltpu.VMEM((CHUNK,), jnp.float32),
            [pltpu.VMEM((CHUNK,), jnp.int32) for _ in range(8)],
            [pltpu.VMEM((CHUNK,), jnp.float32) for _ in range(8)],
            pltpu.VMEM((CHUNK,), jnp.float32),
            pltpu.SemaphoreType.DMA,
        ],
    )(_body)
    return f(src_flat, flow_flat)


def kernel(source, flow_field):
    src_flat = source.reshape(N)
    flow_flat = flow_field.reshape(3 * N)
    out = _run(src_flat, flow_flat)
    return out.reshape(source.shape)


# speedup vs baseline: 1.8816x; 1.1387x over previous
import functools

import jax
import jax.numpy as jnp
from jax import lax
from jax.experimental import pallas as pl
from jax.experimental.pallas import tpu as pltpu
from jax.experimental.pallas import tpu_sc as plsc

D, H, W = 160, 192, 224
N = D * H * W
HW = H * W
NC, NS = 2, 16
NW = NC * NS
ROWS_PER_TILE = (N // W) // NW
CHUNK_ROWS = 8
CHUNK = CHUNK_ROWS * W
NCHUNK = ROWS_PER_TILE // CHUNK_ROWS
PER_TILE = ROWS_PER_TILE * W
VPR = W // 16


def _floor(c):
    t = c.astype(jnp.int32)
    tf = t.astype(jnp.float32)
    adj = tf > c
    fi = t - jnp.where(adj, 1, 0)
    ff = tf - jnp.where(adj, 1.0, 0.0)
    return fi, c - ff


def _dim(c, size):
    fi, fr = _floor(c)
    c0 = jnp.clip(fi, 0, size - 1)
    c1 = jnp.clip(fi + 1, 0, size - 1)
    m0 = (fi >= 0) & (fi < size)
    m1 = (fi >= -1) & (fi < size - 1)
    w0 = jnp.where(m0, 1.0 - fr, 0.0)
    w1 = jnp.where(m1, fr, 0.0)
    return c0, c1, w0, w1


def _body(src_hbm, flow_hbm, out_hbm,
          fz, fy, fx, idx, w, g, ob, sflow, sg):
    wid = lax.axis_index("s") * NC + lax.axis_index("c")
    iota_f = lax.iota(jnp.int32, 16).astype(jnp.float32)
    xvecs = [iota_f + jnp.float32(16 * v) for v in range(VPR)]
    tbase = wid * PER_TILE
    grow_t = wid * ROWS_PER_TILE

    def flow_start(k, p):
        base = tbase + k * CHUNK
        pltpu.async_copy(flow_hbm.at[pl.ds(base, CHUNK)], fz[p], sflow)
        pltpu.async_copy(flow_hbm.at[pl.ds(N + base, CHUNK)], fy[p], sflow)
        pltpu.async_copy(flow_hbm.at[pl.ds(2 * N + base, CHUNK)], fx[p], sflow)

    def flow_wait(p):
        pltpu.make_async_copy(flow_hbm.at[pl.ds(0, CHUNK)], fz[p], sflow).wait()
        pltpu.make_async_copy(flow_hbm.at[pl.ds(0, CHUNK)], fy[p], sflow).wait()
        pltpu.make_async_copy(flow_hbm.at[pl.ds(0, CHUNK)], fx[p], sflow).wait()

    def gather_start(p):
        for c in range(8):
            pltpu.async_copy(src_hbm.at[idx[p][c]], g[p][c], sg)

    def gather_wait(p):
        for c in range(8):
            pltpu.make_async_copy(src_hbm.at[idx[p][c]], g[p][c], sg).wait()

    def pass_a(k, p):
        grow0 = grow_t + k * CHUNK_ROWS

        def row(r, c2):
            grow = grow0 + r
            zf = (grow // H).astype(jnp.float32)
            yf = (grow % H).astype(jnp.float32)
            off = r * W
            for v in range(VPR):
                o = off + v * 16
                cz = fz[p][pl.ds(o, 16)] + zf
                cy = fy[p][pl.ds(o, 16)] + yf
                cx = fx[p][pl.ds(o, 16)] + xvecs[v]
                z0, z1, wz0, wz1 = _dim(cz, D)
                y0, y1, wy0, wy1 = _dim(cy, H)
                x0, x1, wx0, wx1 = _dim(cx, W)
                r00 = z0 * HW + y0 * W
                r01 = z0 * HW + y1 * W
                r10 = z1 * HW + y0 * W
                r11 = z1 * HW + y1 * W
                ds = pl.ds(o, 16)
                idx[p][0][ds] = r00 + x0
                idx[p][1][ds] = r00 + x1
                idx[p][2][ds] = r01 + x0
                idx[p][3][ds] = r01 + x1
                idx[p][4][ds] = r10 + x0
                idx[p][5][ds] = r10 + x1
                idx[p][6][ds] = r11 + x0
                idx[p][7][ds] = r11 + x1
                w00 = wz0 * wy0
                w01 = wz0 * wy1
                w10 = wz1 * wy0
                w11 = wz1 * wy1
                w[p][0][ds] = w00 * wx0
                w[p][1][ds] = w00 * wx1
                w[p][2][ds] = w01 * wx0
                w[p][3][ds] = w01 * wx1
                w[p][4][ds] = w10 * wx0
                w[p][5][ds] = w10 * wx1
                w[p][6][ds] = w11 * wx0
                w[p][7][ds] = w11 * wx1
            return c2

        lax.fori_loop(0, CHUNK_ROWS, row, 0)

    def pass_b(k, p):
        def vec(i, c2):
            ds = pl.ds(i * 16, 16)
            acc = w[p][0][ds] * g[p][0][ds]
            for c in range(1, 8):
                acc = acc + w[p][c][ds] * g[p][c][ds]
            ob[p][ds] = acc
            return c2

        lax.fori_loop(0, CHUNK // 16, vec, 0)
        pltpu.sync_copy(ob[p], out_hbm.at[pl.ds(tbase + k * CHUNK, CHUNK)])

    flow_start(0, 0)
    flow_wait(0)
    pass_a(0, 0)
    gather_start(0)
    flow_start(1, 1)

    def body(j, carry):
        k0 = 2 * j
        flow_wait(1)
        pass_a(k0 + 1, 1)
        gather_start(1)
        flow_start(k0 + 2, 0)
        gather_wait(0)
        pass_b(k0, 0)
        flow_wait(0)
        pass_a(k0 + 2, 0)
        gather_start(0)
        flow_start(k0 + 3, 1)
        gather_wait(1)
        pass_b(k0 + 1, 1)
        return carry

    lax.fori_loop(0, NCHUNK // 2 - 1, body, 0)

    flow_wait(1)
    pass_a(NCHUNK - 1, 1)
    gather_start(1)
    gather_wait(0)
    pass_b(NCHUNK - 2, 0)
    gather_wait(1)
    pass_b(NCHUNK - 1, 1)


@jax.jit
def _run(src_flat, flow_flat):
    mesh = plsc.VectorSubcoreMesh(core_axis_name="c", subcore_axis_name="s")
    vf = lambda: pltpu.VMEM((CHUNK,), jnp.float32)
    vi = lambda: pltpu.VMEM((CHUNK,), jnp.int32)
    f = functools.partial(
        pl.kernel,
        out_type=jax.ShapeDtypeStruct((N,), jnp.float32),
        mesh=mesh,
        scratch_types=[
            [vf(), vf()],
            [vf(), vf()],
            [vf(), vf()],
            [[vi() for _ in range(8)] for _ in range(2)],
            [[vf() for _ in range(8)] for _ in range(2)],
            [[vf() for _ in range(8)] for _ in range(2)],
            [vf(), vf()],
            pltpu.SemaphoreType.DMA,
            pltpu.SemaphoreType.DMA,
        ],
    )(_body)
    return f(src_flat, flow_flat)


def kernel(source, flow_field):
    src_flat = source.reshape(N)
    flow_flat = flow_field.reshape(3 * N)
    out = _run(src_flat, flow_flat)
    return out.reshape(source.shape)


# speedup vs baseline: 4.8657x; 2.5860x over previous
import functools

import jax
import jax.numpy as jnp
from jax import lax
from jax.experimental import pallas as pl
from jax.experimental.pallas import tpu as pltpu
from jax.experimental.pallas import tpu_sc as plsc

D, H, W = 160, 192, 224
N = D * H * W
HW = H * W
NC, NS = 2, 16
NW = NC * NS

BZ, BY = 2, 12
HALO = 6
SNZ, SNY = BZ + 2 * HALO + 1, BY + 2 * HALO + 1
SNYW = SNY * W
SLABW = SNZ * SNYW
NB_Z, NB_Y = D // BZ, H // BY
BLK_PER_TILE = (NB_Z * NB_Y) // NW
BLKV = BZ * BY * W
ROWV = BY * W
VPR = W // 16


def _floor(c):
    t = c.astype(jnp.int32)
    tf = t.astype(jnp.float32)
    adj = tf > c
    fi = t - jnp.where(adj, 1, 0)
    ff = tf - jnp.where(adj, 1.0, 0.0)
    return fi, c - ff


def _dim(c, size):
    fi, fr = _floor(c)
    c0 = jnp.clip(fi, 0, size - 1)
    c1 = jnp.clip(fi + 1, 0, size - 1)
    m0 = (fi >= 0) & (fi < size)
    m1 = (fi >= -1) & (fi < size - 1)
    w0 = jnp.where(m0, 1.0 - fr, 0.0)
    w1 = jnp.where(m1, fr, 0.0)
    return c0, c1, w0, w1


def _body(src_hbm, flow_hbm, out_hbm, slab, flz, fly, flx, ob, fb,
          sdma, sout, sfb):
    wid = lax.axis_index("s") * NC + lax.axis_index("c")
    iota_f = lax.iota(jnp.int32, 16).astype(jnp.float32)

    def blk_body(blk, carry):
        b = wid * BLK_PER_TILE + blk
        bz = b // NB_Y
        by = b % NB_Y
        z0b = bz * BZ
        y0b = by * BY
        szlo = jnp.clip(z0b - HALO, 0, D - SNZ)
        sylo = jnp.clip(y0b - HALO, 0, H - SNY)

        for i in range(SNZ):
            off = ((szlo + i) * H + sylo) * W
            pltpu.async_copy(src_hbm.at[pl.ds(off, SNYW)],
                             slab.at[pl.ds(i * SNYW, SNYW)], sdma)
        for zz in range(BZ):
            off = ((z0b + zz) * H + y0b) * W
            pltpu.async_copy(flow_hbm.at[pl.ds(off, ROWV)],
                             flz.at[pl.ds(zz * ROWV, ROWV)], sdma)
            pltpu.async_copy(flow_hbm.at[pl.ds(N + off, ROWV)],
                             fly.at[pl.ds(zz * ROWV, ROWV)], sdma)
            pltpu.async_copy(flow_hbm.at[pl.ds(2 * N + off, ROWV)],
                             flx.at[pl.ds(zz * ROWV, ROWV)], sdma)

        @pl.when(blk > 0)
        def _():
            for _ in range(BZ):
                pltpu.make_async_copy(
                    ob.at[pl.ds(0, ROWV)],
                    out_hbm.at[pl.ds(0, ROWV)], sout).wait()

        for i in range(SNZ):
            pltpu.make_async_copy(src_hbm.at[pl.ds(0, SNYW)],
                                  slab.at[pl.ds(0, SNYW)], sdma).wait()
        for _ in range(BZ * 3):
            pltpu.make_async_copy(flow_hbm.at[pl.ds(0, ROWV)],
                                  flz.at[pl.ds(0, ROWV)], sdma).wait()

        def row(rr, c2):
            zz = rr // BY
            yy = rr % BY
            zf = (z0b + zz).astype(jnp.float32)
            yf = (y0b + yy).astype(jnp.float32)

            def vec(v, c3):
                o = rr * W + v * 16
                dsl = pl.ds(o, 16)
                xv = iota_f + (v * 16).astype(jnp.float32)
                cz = flz[dsl] + zf
                cy = fly[dsl] + yf
                cx = flx[dsl] + xv
                z0, z1, wz0, wz1 = _dim(cz, D)
                y0, y1, wy0, wy1 = _dim(cy, H)
                x0, x1, wx0, wx1 = _dim(cx, W)
                zl0 = z0 - szlo
                zl1 = z1 - szlo
                yl0 = y0 - sylo
                yl1 = y1 - sylo
                inz0 = (zl0 >= 0) & (zl0 < SNZ)
                inz1 = (zl1 >= 0) & (zl1 < SNZ)
                iny0 = (yl0 >= 0) & (yl0 < SNY)
                iny1 = (yl1 >= 0) & (yl1 < SNY)
                in00 = inz0 & iny0
                in01 = inz0 & iny1
                in10 = inz1 & iny0
                in11 = inz1 & iny1
                allok = jnp.all(in00 & in01 & in10 & in11)
                rb00 = zl0 * SNYW + yl0 * W
                rb01 = zl0 * SNYW + yl1 * W
                rb10 = zl1 * SNYW + yl0 * W
                rb11 = zl1 * SNYW + yl1 * W
                zero = jnp.zeros((16,), jnp.int32)
                l0 = jnp.where(in00, rb00 + x0, zero)
                l1 = jnp.where(in00, rb00 + x1, zero)
                l2 = jnp.where(in01, rb01 + x0, zero)
                l3 = jnp.where(in01, rb01 + x1, zero)
                l4 = jnp.where(in10, rb10 + x0, zero)
                l5 = jnp.where(in10, rb10 + x1, zero)
                l6 = jnp.where(in11, rb11 + x0, zero)
                l7 = jnp.where(in11, rb11 + x1, zero)
                w00 = wz0 * wy0
                w01 = wz0 * wy1
                w10 = wz1 * wy0
                w11 = wz1 * wy1
                wv = (w00 * wx0, w00 * wx1, w01 * wx0, w01 * wx1,
                      w10 * wx0, w10 * wx1, w11 * wx0, w11 * wx1)
                acc = wv[0] * plsc.load_gather(slab, [l0])
                acc = acc + wv[1] * plsc.load_gather(slab, [l1])
                acc = acc + wv[2] * plsc.load_gather(slab, [l2])
                acc = acc + wv[3] * plsc.load_gather(slab, [l3])
                acc = acc + wv[4] * plsc.load_gather(slab, [l4])
                acc = acc + wv[5] * plsc.load_gather(slab, [l5])
                acc = acc + wv[6] * plsc.load_gather(slab, [l6])
                acc = acc + wv[7] * plsc.load_gather(slab, [l7])
                ob[dsl] = acc

                @pl.when(jnp.logical_not(allok))
                def _():
                    gb00 = z0 * HW + y0 * W
                    gb01 = z0 * HW + y1 * W
                    gb10 = z1 * HW + y0 * W
                    gb11 = z1 * HW + y1 * W
                    gidx = (gb00 + x0, gb00 + x1, gb01 + x0, gb01 + x1,
                            gb10 + x0, gb10 + x1, gb11 + x0, gb11 + x1)
                    cps = [pltpu.async_copy(src_hbm.at[gidx[c]], fb[c], sfb)
                           for c in range(8)]
                    for cp in cps:
                        cp.wait()
                    acc2 = wv[0] * fb[0][...]
                    for c in range(1, 8):
                        acc2 = acc2 + wv[c] * fb[c][...]
                    ob[dsl] = acc2

                return c3

            lax.fori_loop(0, VPR, vec, 0)
            return c2

        lax.fori_loop(0, BZ * BY, row, 0)

        for zz in range(BZ):
            off = ((z0b + zz) * H + y0b) * W
            pltpu.async_copy(ob.at[pl.ds(zz * ROWV, ROWV)],
                             out_hbm.at[pl.ds(off, ROWV)], sout)
        return carry

    lax.fori_loop(0, BLK_PER_TILE, blk_body, 0)
    for _ in range(BZ):
        pltpu.make_async_copy(ob.at[pl.ds(0, ROWV)],
                              out_hbm.at[pl.ds(0, ROWV)], sout).wait()


@jax.jit
def _run(src_flat, flow_flat):
    mesh = plsc.VectorSubcoreMesh(core_axis_name="c", subcore_axis_name="s")
    f = functools.partial(
        pl.kernel,
        out_type=jax.ShapeDtypeStruct((N,), jnp.float32),
        mesh=mesh,
        compiler_params=pltpu.CompilerParams(needs_layout_passes=False),
        scratch_types=[
            pltpu.VMEM((SLABW,), jnp.float32),
            pltpu.VMEM((BLKV,), jnp.float32),
            pltpu.VMEM((BLKV,), jnp.float32),
            pltpu.VMEM((BLKV,), jnp.float32),
            pltpu.VMEM((BLKV,), jnp.float32),
            [pltpu.VMEM((16,), jnp.float32) for _ in range(8)],
            pltpu.SemaphoreType.DMA,
            pltpu.SemaphoreType.DMA,
            pltpu.SemaphoreType.DMA,
        ],
    )(_body)
    return f(src_flat, flow_flat)


def kernel(source, flow_field):
    src_flat = source.reshape(N)
    flow_flat = flow_field.reshape(3 * N)
    out = _run(src_flat, flow_flat)
    return out.reshape(source.shape)


# speedup vs baseline: 6.3558x; 1.3062x over previous
import functools

import jax
import jax.numpy as jnp
from jax import lax
from jax.experimental import pallas as pl
from jax.experimental.pallas import tpu as pltpu
from jax.experimental.pallas import tpu_sc as plsc

D, H, W = 160, 192, 224
N = D * H * W
HW = H * W
NC, NS = 2, 16
NW = NC * NS

BZ, BY = 2, 12
HALO = 6
SNZ, SNY = BZ + 2 * HALO + 1, BY + 2 * HALO + 1
SNYW = SNY * W
SLABW = SNZ * SNYW
NB_Z, NB_Y = D // BZ, H // BY
BLK_PER_TILE = (NB_Z * NB_Y) // NW
BLKV = BZ * BY * W
ROWV = BY * W
VPR = W // 16


def _floor(c):
    t = c.astype(jnp.int32)
    tf = t.astype(jnp.float32)
    adj = tf > c
    fi = t - jnp.where(adj, 1, 0)
    ff = tf - jnp.where(adj, 1.0, 0.0)
    return fi, c - ff


def _dim(c, size):
    fi, fr = _floor(c)
    fi1 = fi + 1
    c0 = jnp.clip(fi, 0, size - 1)
    c1 = jnp.clip(fi1, 0, size - 1)
    m0 = fi.astype(jnp.uint32) < jnp.uint32(size)
    m1 = fi1.astype(jnp.uint32) < jnp.uint32(size)
    w0 = jnp.where(m0, 1.0 - fr, 0.0)
    w1 = jnp.where(m1, fr, 0.0)
    return c0, c1, w0, w1


def _corners(cz, cy, cx, zf_shift, yf_shift, zstride, ystride):
    z0, z1, wz0, wz1 = _dim(cz, D)
    y0, y1, wy0, wy1 = _dim(cy, H)
    x0, x1, wx0, wx1 = _dim(cx, W)
    zl0 = z0 - zf_shift
    zl1 = z1 - zf_shift
    yl0 = y0 - yf_shift
    yl1 = y1 - yf_shift
    rb00 = zl0 * zstride + yl0 * ystride
    rb01 = zl0 * zstride + yl1 * ystride
    rb10 = zl1 * zstride + yl0 * ystride
    rb11 = zl1 * zstride + yl1 * ystride
    w00 = wz0 * wy0
    w01 = wz0 * wy1
    w10 = wz1 * wy0
    w11 = wz1 * wy1
    wv = (w00 * wx0, w00 * wx1, w01 * wx0, w01 * wx1,
          w10 * wx0, w10 * wx1, w11 * wx0, w11 * wx1)
    return (zl0, zl1, yl0, yl1), (rb00, rb01, rb10, rb11), (x0, x1), wv


def _body(src_hbm, flow_hbm, out_hbm, slab, flz, fly, flx, ob, fb,
          sdma, sout, sfb):
    wid = lax.axis_index("s") * NC + lax.axis_index("c")
    iota_f = lax.iota(jnp.int32, 16).astype(jnp.float32)

    def blk_body(blk, carry):
        b = wid * BLK_PER_TILE + blk
        bz = b // NB_Y
        by = b % NB_Y
        z0b = bz * BZ
        y0b = by * BY
        szlo = jnp.clip(z0b - HALO, 0, D - SNZ)
        sylo = jnp.clip(y0b - HALO, 0, H - SNY)

        for i in range(SNZ):
            off = ((szlo + i) * H + sylo) * W
            pltpu.async_copy(src_hbm.at[pl.ds(off, SNYW)],
                             slab.at[pl.ds(i * SNYW, SNYW)], sdma)
        for zz in range(BZ):
            off = ((z0b + zz) * H + y0b) * W
            pltpu.async_copy(flow_hbm.at[pl.ds(off, ROWV)],
                             flz.at[pl.ds(zz * ROWV, ROWV)], sdma)
            pltpu.async_copy(flow_hbm.at[pl.ds(N + off, ROWV)],
                             fly.at[pl.ds(zz * ROWV, ROWV)], sdma)
            pltpu.async_copy(flow_hbm.at[pl.ds(2 * N + off, ROWV)],
                             flx.at[pl.ds(zz * ROWV, ROWV)], sdma)

        @pl.when(blk > 0)
        def _():
            for _ in range(BZ):
                pltpu.make_async_copy(
                    ob.at[pl.ds(0, ROWV)],
                    out_hbm.at[pl.ds(0, ROWV)], sout).wait()

        for i in range(SNZ):
            pltpu.make_async_copy(src_hbm.at[pl.ds(0, SNYW)],
                                  slab.at[pl.ds(0, SNYW)], sdma).wait()
        for _ in range(BZ * 3):
            pltpu.make_async_copy(flow_hbm.at[pl.ds(0, ROWV)],
                                  flz.at[pl.ds(0, ROWV)], sdma).wait()

        def row(rr, c2):
            zz = rr // BY
            yy = rr % BY
            zf = (z0b + zz).astype(jnp.float32)
            yf = (y0b + yy).astype(jnp.float32)
            o0 = rr * W

            def vec(v, okacc):
                dsl = pl.ds(o0 + v * 16, 16)
                xv = iota_f + (v * 16).astype(jnp.float32)
                cz = flz[dsl] + zf
                cy = fly[dsl] + yf
                cx = flx[dsl] + xv
                (zl0, zl1, yl0, yl1), rbs, (x0, x1), wv = _corners(
                    cz, cy, cx, szlo, sylo, SNYW, W)
                inz0 = zl0.astype(jnp.uint32) < jnp.uint32(SNZ)
                inz1 = zl1.astype(jnp.uint32) < jnp.uint32(SNZ)
                iny0 = yl0.astype(jnp.uint32) < jnp.uint32(SNY)
                iny1 = yl1.astype(jnp.uint32) < jnp.uint32(SNY)
                ins = (inz0 & iny0, inz0 & iny1, inz1 & iny0, inz1 & iny1)
                zero = jnp.zeros((16,), jnp.int32)
                acc = None
                for q in range(4):
                    lq0 = jnp.where(ins[q], rbs[q] + x0, zero)
                    lq1 = jnp.where(ins[q], rbs[q] + x1, zero)
                    t = (wv[2 * q] * plsc.load_gather(slab, [lq0]) +
                         wv[2 * q + 1] * plsc.load_gather(slab, [lq1]))
                    acc = t if acc is None else acc + t
                ob[dsl] = acc
                ok4 = ins[0] & ins[1] & ins[2] & ins[3]
                return okacc & jnp.where(ok4, 1, zero)

            okv = plsc.parallel_loop(0, VPR, carry=jnp.ones((16,), jnp.int32))(vec)

            @pl.when(jnp.any(okv == 0))
            def _():
                def fvec(v, c3):
                    dsl = pl.ds(o0 + v * 16, 16)
                    xv = iota_f + (v * 16).astype(jnp.float32)
                    cz = flz[dsl] + zf
                    cy = fly[dsl] + yf
                    cx = flx[dsl] + xv
                    _, gbs, (x0, x1), wv = _corners(
                        cz, cy, cx, 0, 0, HW, W)
                    cps = []
                    for q in range(4):
                        cps.append(pltpu.async_copy(
                            src_hbm.at[gbs[q] + x0], fb[2 * q], sfb))
                        cps.append(pltpu.async_copy(
                            src_hbm.at[gbs[q] + x1], fb[2 * q + 1], sfb))
                    for cp in cps:
                        cp.wait()
                    acc2 = wv[0] * fb[0][...]
                    for c in range(1, 8):
                        acc2 = acc2 + wv[c] * fb[c][...]
                    ob[dsl] = acc2
                    return c3

                lax.fori_loop(0, VPR, fvec, 0)

            return c2

        lax.fori_loop(0, BZ * BY, row, 0)

        for zz in range(BZ):
            off = ((z0b + zz) * H + y0b) * W
            pltpu.async_copy(ob.at[pl.ds(zz * ROWV, ROWV)],
                             out_hbm.at[pl.ds(off, ROWV)], sout)
        return carry

    lax.fori_loop(0, BLK_PER_TILE, blk_body, 0)
    for _ in range(BZ):
        pltpu.make_async_copy(ob.at[pl.ds(0, ROWV)],
                              out_hbm.at[pl.ds(0, ROWV)], sout).wait()


@jax.jit
def _run(src_flat, flow_flat):
    mesh = plsc.VectorSubcoreMesh(core_axis_name="c", subcore_axis_name="s")
    f = functools.partial(
        pl.kernel,
        out_type=jax.ShapeDtypeStruct((N,), jnp.float32),
        mesh=mesh,
        compiler_params=pltpu.CompilerParams(needs_layout_passes=False),
        scratch_types=[
            pltpu.VMEM((SLABW,), jnp.float32),
            pltpu.VMEM((BLKV,), jnp.float32),
            pltpu.VMEM((BLKV,), jnp.float32),
            pltpu.VMEM((BLKV,), jnp.float32),
            pltpu.VMEM((BLKV,), jnp.float32),
            [pltpu.VMEM((16,), jnp.float32) for _ in range(8)],
            pltpu.SemaphoreType.DMA,
            pltpu.SemaphoreType.DMA,
            pltpu.SemaphoreType.DMA,
        ],
    )(_body)
    return f(src_flat, flow_flat)


def kernel(source, flow_field):
    src_flat = source.reshape(N)
    flow_flat = flow_field.reshape(3 * N)
    out = _run(src_flat, flow_flat)
    return out.reshape(source.shape)


# speedup vs baseline: 7.0820x; 1.1143x over previous
import functools

import jax
import jax.numpy as jnp
from jax import lax
from jax.experimental import pallas as pl
from jax.experimental.pallas import tpu as pltpu
from jax.experimental.pallas import tpu_sc as plsc

D, H, W = 160, 192, 224
N = D * H * W
HW = H * W
NC, NS = 2, 16
NW = NC * NS

BZ, BY = 5, 6
HALO = 6
SNZ, SNY = BZ + 2 * HALO + 1, BY + 2 * HALO + 1
SNYW = SNY * W
SLABW = SNZ * SNYW
NB_Z, NB_Y = D // BZ, H // BY
BLKV = BZ * BY * W
ROWV = BY * W
VPR = W // 16


def _floor(c):
    t = c.astype(jnp.int32)
    tf = t.astype(jnp.float32)
    adj = tf > c
    fi = t - jnp.where(adj, 1, 0)
    ff = tf - jnp.where(adj, 1.0, 0.0)
    return fi, c - ff


def _dim(c, size):
    fi, fr = _floor(c)
    fi1 = fi + 1
    c0 = jnp.clip(fi, 0, size - 1)
    c1 = jnp.clip(fi1, 0, size - 1)
    m0 = fi.astype(jnp.uint32) < jnp.uint32(size)
    m1 = fi1.astype(jnp.uint32) < jnp.uint32(size)
    w0 = jnp.where(m0, 1.0 - fr, 0.0)
    w1 = jnp.where(m1, fr, 0.0)
    return c0, c1, w0, w1


def _corners(cz, cy, cx, zf_shift, yf_shift):
    z0, z1, wz0, wz1 = _dim(cz, D)
    y0, y1, wy0, wy1 = _dim(cy, H)
    x0, x1, wx0, wx1 = _dim(cx, W)
    zl0 = z0 - zf_shift
    zl1 = z1 - zf_shift
    yl0 = y0 - yf_shift
    yl1 = y1 - yf_shift
    w00 = wz0 * wy0
    w01 = wz0 * wy1
    w10 = wz1 * wy0
    w11 = wz1 * wy1
    wv = (w00 * wx0, w00 * wx1, w01 * wx0, w01 * wx1,
          w10 * wx0, w10 * wx1, w11 * wx0, w11 * wx1)
    return (zl0, zl1, yl0, yl1), (x0, x1), wv


def _body(src_hbm, flow_hbm, out_hbm, slab, flz, fly, flx, ob, fb,
          sdma, sout, sfb):
    wid = lax.axis_index("s") * NC + lax.axis_index("c")
    iota_f = lax.iota(jnp.int32, 16).astype(jnp.float32)
    y0b = wid * BY
    sylo = jnp.clip(y0b - HALO, 0, H - SNY)

    def blk_body(blk, prev_end):
        z0b = blk * BZ
        szlo = jnp.clip(z0b - HALO, 0, D - SNZ)
        bs = szlo % SNZ

        conds = []
        for i in range(SNZ):
            zg = szlo + i
            cond = zg >= prev_end
            conds.append(cond)

            @pl.when(cond)
            def _(zg=zg):
                slot = zg % SNZ
                off = (zg * H + sylo) * W
                pltpu.async_copy(src_hbm.at[pl.ds(off, SNYW)],
                                 slab.at[pl.ds(slot * SNYW, SNYW)], sdma)

        for zz in range(BZ):
            off = ((z0b + zz) * H + y0b) * W
            pltpu.async_copy(flow_hbm.at[pl.ds(off, ROWV)],
                             flz.at[pl.ds(zz * ROWV, ROWV)], sdma)
            pltpu.async_copy(flow_hbm.at[pl.ds(N + off, ROWV)],
                             fly.at[pl.ds(zz * ROWV, ROWV)], sdma)
            pltpu.async_copy(flow_hbm.at[pl.ds(2 * N + off, ROWV)],
                             flx.at[pl.ds(zz * ROWV, ROWV)], sdma)

        @pl.when(blk > 0)
        def _():
            for _ in range(BZ):
                pltpu.make_async_copy(
                    ob.at[pl.ds(0, ROWV)],
                    out_hbm.at[pl.ds(0, ROWV)], sout).wait()

        for i in range(SNZ):
            @pl.when(conds[i])
            def _():
                pltpu.make_async_copy(src_hbm.at[pl.ds(0, SNYW)],
                                      slab.at[pl.ds(0, SNYW)], sdma).wait()
        for _ in range(BZ * 3):
            pltpu.make_async_copy(flow_hbm.at[pl.ds(0, ROWV)],
                                  flz.at[pl.ds(0, ROWV)], sdma).wait()

        def row(rr, c2):
            zz = rr // BY
            yy = rr % BY
            zf = (z0b + zz).astype(jnp.float32)
            yf = (y0b + yy).astype(jnp.float32)
            o0 = rr * W

            def vec(v, okacc):
                dsl = pl.ds(o0 + v * 16, 16)
                xv = iota_f + (v * 16).astype(jnp.float32)
                cz = flz[dsl] + zf
                cy = fly[dsl] + yf
                cx = flx[dsl] + xv
                (zl0, zl1, yl0, yl1), (x0, x1), wv = _corners(
                    cz, cy, cx, szlo, sylo)
                inz0 = zl0.astype(jnp.uint32) < jnp.uint32(SNZ)
                inz1 = zl1.astype(jnp.uint32) < jnp.uint32(SNZ)
                iny0 = yl0.astype(jnp.uint32) < jnp.uint32(SNY)
                iny1 = yl1.astype(jnp.uint32) < jnp.uint32(SNY)
                s0 = zl0 + bs
                s0 = s0 - jnp.where(s0 >= SNZ, SNZ, 0)
                s1 = zl1 + bs
                s1 = s1 - jnp.where(s1 >= SNZ, SNZ, 0)
                rb00 = s0 * SNYW + yl0 * W
                rb01 = s0 * SNYW + yl1 * W
                rb10 = s1 * SNYW + yl0 * W
                rb11 = s1 * SNYW + yl1 * W
                rbs = (rb00, rb01, rb10, rb11)
                ins = (inz0 & iny0, inz0 & iny1, inz1 & iny0, inz1 & iny1)
                zero = jnp.zeros((16,), jnp.int32)
                acc = None
                for q in range(4):
                    lq0 = jnp.where(ins[q], rbs[q] + x0, zero)
                    lq1 = jnp.where(ins[q], rbs[q] + x1, zero)
                    t = (wv[2 * q] * plsc.load_gather(slab, [lq0]) +
                         wv[2 * q + 1] * plsc.load_gather(slab, [lq1]))
                    acc = t if acc is None else acc + t
                ob[dsl] = acc
                ok4 = ins[0] & ins[1] & ins[2] & ins[3]
                return okacc & jnp.where(ok4, 1, zero)

            okv = plsc.parallel_loop(0, VPR, carry=jnp.ones((16,), jnp.int32))(vec)

            @pl.when(jnp.any(okv == 0))
            def _():
                def fvec(v, c3):
                    dsl = pl.ds(o0 + v * 16, 16)
                    xv = iota_f + (v * 16).astype(jnp.float32)
                    cz = flz[dsl] + zf
                    cy = fly[dsl] + yf
                    cx = flx[dsl] + xv
                    (gz0, gz1, gy0, gy1), (x0, x1), wv = _corners(
                        cz, cy, cx, 0, 0)
                    gbs = (gz0 * HW + gy0 * W, gz0 * HW + gy1 * W,
                           gz1 * HW + gy0 * W, gz1 * HW + gy1 * W)
                    cps = []
                    for q in range(4):
                        cps.append(pltpu.async_copy(
                            src_hbm.at[gbs[q] + x0], fb[2 * q], sfb))
                        cps.append(pltpu.async_copy(
                            src_hbm.at[gbs[q] + x1], fb[2 * q + 1], sfb))
                    for cp in cps:
                        cp.wait()
                    acc2 = wv[0] * fb[0][...]
                    for c in range(1, 8):
                        acc2 = acc2 + wv[c] * fb[c][...]
                    ob[dsl] = acc2
                    return c3

                lax.fori_loop(0, VPR, fvec, 0)

            return c2

        lax.fori_loop(0, BZ * BY, row, 0)

        for zz in range(BZ):
            off = ((z0b + zz) * H + y0b) * W
            pltpu.async_copy(ob.at[pl.ds(zz * ROWV, ROWV)],
                             out_hbm.at[pl.ds(off, ROWV)], sout)
        return szlo + SNZ

    lax.fori_loop(0, NB_Z, blk_body, jnp.int32(0))
    for _ in range(BZ):
        pltpu.make_async_copy(ob.at[pl.ds(0, ROWV)],
                              out_hbm.at[pl.ds(0, ROWV)], sout).wait()


@jax.jit
def _run(src_flat, flow_flat):
    mesh = plsc.VectorSubcoreMesh(core_axis_name="c", subcore_axis_name="s")
    f = functools.partial(
        pl.kernel,
        out_type=jax.ShapeDtypeStruct((N,), jnp.float32),
        mesh=mesh,
        compiler_params=pltpu.CompilerParams(needs_layout_passes=False),
        scratch_types=[
            pltpu.VMEM((SLABW,), jnp.float32),
            pltpu.VMEM((BLKV,), jnp.float32),
            pltpu.VMEM((BLKV,), jnp.float32),
            pltpu.VMEM((BLKV,), jnp.float32),
            pltpu.VMEM((BLKV,), jnp.float32),
            [pltpu.VMEM((16,), jnp.float32) for _ in range(8)],
            pltpu.SemaphoreType.DMA,
            pltpu.SemaphoreType.DMA,
            pltpu.SemaphoreType.DMA,
        ],
    )(_body)
    return f(src_flat, flow_flat)


def kernel(source, flow_field):
    src_flat = source.reshape(N)
    flow_flat = flow_field.reshape(3 * N)
    out = _run(src_flat, flow_flat)
    return out.reshape(source.shape)
